# Initial kernel scaffold; baseline (speedup 1.0000x reference)
import jax
import jax.numpy as jnp
from jax.experimental import pallas as pl


def kernel(x, img, index_x, index_y, proj_x, proj_y):
    raise NotImplementedError("write your pallas kernel here")


# speedup vs baseline: 28.6769x; 28.6769x over previous
import functools

import jax
import jax.numpy as jnp
from jax import lax
from jax.experimental import pallas as pl
from jax.experimental.pallas import tpu as pltpu
from jax.experimental.pallas import tpu_sc as plsc

C = 64
L = 100000
X_H = 512
X_W = 512
IMG_H = 128
IMG_W = 2048

N_TILES = 16
P = 6272
LP = N_TILES * P
ROWS = P // 16
PLANE = X_H * X_W
SLICE = PLANE // N_TILES
CH_PER_CORE = C // 2


def _sc_kernel(x_hbm, img_hbm, px_hbm, py_hbm, ix_hbm, iy_hbm, out_hbm,
               ta, tb, pidx_v, didx_v, gidx_v, vals_v, plane_s, sem):
    cid = lax.axis_index("c")
    sid = lax.axis_index("s")

    pltpu.sync_copy(px_hbm.at[sid], ta)
    pltpu.sync_copy(py_hbm.at[sid], tb)

    def body_p(j, carry):
        s = pl.ds(j * 16, 16)
        pidx_v[s] = tb[s] * IMG_W + ta[s]
        return carry

    lax.fori_loop(0, ROWS, body_p, 0)

    pltpu.sync_copy(ix_hbm.at[sid], ta)
    pltpu.sync_copy(iy_hbm.at[sid], tb)

    def body_d(j, carry):
        s = pl.ds(j * 16, 16)
        didx_v[s] = ta[s] * X_W + tb[s]
        return carry

    lax.fori_loop(0, ROWS, body_d, 0)

    def chan_body(k, carry):
        c = cid * CH_PER_CORE + k
        pltpu.sync_copy(x_hbm.at[c, pl.ds(sid * SLICE, SLICE)],
                        plane_s.at[pl.ds(sid * SLICE, SLICE)])
        plsc.subcore_barrier()

        base = c * PLANE

        def body_g(j, carry2):
            s = pl.ds(j * 16, 16)
            gidx_v[s] = pidx_v[s] + base
            return carry2

        lax.fori_loop(0, ROWS, body_g, 0)

        pltpu.async_copy(img_hbm.at[gidx_v], vals_v, sem).wait()
        pltpu.sync_copy(vals_v, plane_s.at[didx_v], add=True)
        plsc.subcore_barrier()

        pltpu.sync_copy(plane_s.at[pl.ds(sid * SLICE, SLICE)],
                        out_hbm.at[c, pl.ds(sid * SLICE, SLICE)])
        return carry

    lax.fori_loop(0, CH_PER_CORE, chan_body, 0)


def kernel(x, img, index_x, index_y, proj_x, proj_y):
    pad = LP - L
    px = jnp.concatenate([proj_x, jnp.zeros((pad,), jnp.int32)])
    py = jnp.concatenate([proj_y, jnp.zeros((pad,), jnp.int32)])
    ix = jnp.concatenate([index_x.reshape(-1), jnp.full((pad,), X_H, jnp.int32)])
    iy = jnp.concatenate([index_y.reshape(-1), jnp.zeros((pad,), jnp.int32)])
    px = px.reshape(N_TILES, P)
    py = py.reshape(N_TILES, P)
    ix = ix.reshape(N_TILES, P)
    iy = iy.reshape(N_TILES, P)

    x2 = x.reshape(C, PLANE)
    img_flat = img.reshape(-1)

    mesh = plsc.VectorSubcoreMesh(core_axis_name="c", subcore_axis_name="s")
    kern = functools.partial(
        pl.kernel,
        out_type=jax.ShapeDtypeStruct((C, PLANE), jnp.float32),
        mesh=mesh,
        scratch_types=[
            pltpu.VMEM((P,), jnp.int32),
            pltpu.VMEM((P,), jnp.int32),
            pltpu.VMEM((P,), jnp.int32),
            pltpu.VMEM((P,), jnp.int32),
            pltpu.VMEM((P,), jnp.int32),
            pltpu.VMEM((P,), jnp.float32),
            pltpu.VMEM_SHARED((PLANE + 16,), jnp.float32),
            pltpu.SemaphoreType.DMA,
        ],
    )(_sc_kernel)

    out = kern(x2, img_flat, px, py, ix, iy)
    return out.reshape(C, X_H, X_W)


# speedup vs baseline: 41.2708x; 1.4392x over previous
import functools

import jax
import jax.numpy as jnp
from jax import lax
from jax.experimental import pallas as pl
from jax.experimental.pallas import tpu as pltpu
from jax.experimental.pallas import tpu_sc as plsc

C = 64
L = 100000
X_H = 512
X_W = 512
IMG_H = 128
IMG_W = 2048

N_TILES = 16
P = 6272
LP = N_TILES * P
ROWS = P // 16
PLANE = X_H * X_W
SLICE = PLANE // N_TILES
CH_PER_CORE = C // 2


def _sc_kernel(x_hbm, img_hbm, px_hbm, py_hbm, ix_hbm, iy_hbm, out_hbm,
               ta, tb, pidx_v, didx_v, vals0, vals1, plane0, plane1,
               gsem0, gsem1, psem, osem):
    cid = lax.axis_index("c")
    sid = lax.axis_index("s")

    pltpu.sync_copy(px_hbm.at[sid], ta)
    pltpu.sync_copy(py_hbm.at[sid], tb)

    def body_p(j, carry):
        s = pl.ds(j * 16, 16)
        pidx_v[s] = tb[s] * IMG_W + ta[s]
        return carry

    lax.fori_loop(0, ROWS, body_p, 0)

    pltpu.sync_copy(ix_hbm.at[sid], ta)
    pltpu.sync_copy(iy_hbm.at[sid], tb)

    def body_d(j, carry):
        s = pl.ds(j * 16, 16)
        didx_v[s] = ta[s] * X_W + tb[s]
        return carry

    lax.fori_loop(0, ROWS, body_d, 0)

    def chan(k):
        return cid * CH_PER_CORE + k

    def gather_src(k):
        return img_hbm.at[pl.ds(chan(k) * PLANE, PLANE)].at[pidx_v]

    def plane_slice(buf):
        return buf.at[pl.ds(sid * SLICE, SLICE)]

    def x_slice(k):
        return x_hbm.at[chan(k), pl.ds(sid * SLICE, SLICE)]

    def out_slice(k):
        return out_hbm.at[chan(k), pl.ds(sid * SLICE, SLICE)]

    pltpu.async_copy(gather_src(0), vals0, gsem0)
    pltpu.sync_copy(x_slice(0), plane_slice(plane0))
    plsc.subcore_barrier()

    def chan_body(k, carry):
        b = k % 2
        vals_b = [vals0, vals1]
        plane_b = [plane0, plane1]
        gsem_b = [gsem0, gsem1]

        def cur(refs):
            return lax.cond(b == 0, lambda: 0, lambda: 1)

        def run(b_static):
            vals = vals_b[b_static]
            valsn = vals_b[1 - b_static]
            plane = plane_b[b_static]
            planen = plane_b[1 - b_static]
            gsem = gsem_b[b_static]
            gsemn = gsem_b[1 - b_static]

            @pl.when(k < CH_PER_CORE - 1)
            def _prefetch():
                @pl.when(k >= 1)
                def _drain_out():
                    pltpu.make_async_copy(plane_slice(planen), out_slice(k - 1),
                                          osem).wait()
                pltpu.async_copy(x_slice(k + 1), plane_slice(planen), psem)
                pltpu.async_copy(gather_src(k + 1), valsn, gsemn)

            @pl.when(k == CH_PER_CORE - 1)
            def _drain_out_last():
                @pl.when(k >= 1)
                def _():
                    pltpu.make_async_copy(plane_slice(planen), out_slice(k - 1),
                                          osem).wait()

            pltpu.make_async_copy(gather_src(k), vals, gsem).wait()
            pltpu.sync_copy(vals, plane.at[didx_v], add=True)
            plsc.subcore_barrier()
            pltpu.async_copy(plane_slice(plane), out_slice(k), osem)

            @pl.when(k < CH_PER_CORE - 1)
            def _wait_in():
                pltpu.make_async_copy(x_slice(k + 1), plane_slice(planen),
                                      psem).wait()

        @pl.when(b == 0)
        def _():
            run(0)

        @pl.when(b == 1)
        def _():
            run(1)

        plsc.subcore_barrier()
        return carry

    lax.fori_loop(0, CH_PER_CORE, chan_body, 0)
    b_last = (CH_PER_CORE - 1) % 2
    last_plane = plane1 if b_last else plane0
    pltpu.make_async_copy(plane_slice(last_plane),
                          out_slice(CH_PER_CORE - 1), osem).wait()


def kernel(x, img, index_x, index_y, proj_x, proj_y):
    pad = LP - L
    px = jnp.concatenate([proj_x, jnp.zeros((pad,), jnp.int32)])
    py = jnp.concatenate([proj_y, jnp.zeros((pad,), jnp.int32)])
    ix = jnp.concatenate([index_x.reshape(-1), jnp.full((pad,), X_H, jnp.int32)])
    iy = jnp.concatenate([index_y.reshape(-1), jnp.zeros((pad,), jnp.int32)])
    px = px.reshape(N_TILES, P)
    py = py.reshape(N_TILES, P)
    ix = ix.reshape(N_TILES, P)
    iy = iy.reshape(N_TILES, P)

    x2 = x.reshape(C, PLANE)
    img_flat = img.reshape(-1)

    mesh = plsc.VectorSubcoreMesh(core_axis_name="c", subcore_axis_name="s")
    kern = functools.partial(
        pl.kernel,
        out_type=jax.ShapeDtypeStruct((C, PLANE), jnp.float32),
        mesh=mesh,
        scratch_types=[
            pltpu.VMEM((P,), jnp.int32),
            pltpu.VMEM((P,), jnp.int32),
            pltpu.VMEM((P,), jnp.int32),
            pltpu.VMEM((P,), jnp.int32),
            pltpu.VMEM((P,), jnp.float32),
            pltpu.VMEM((P,), jnp.float32),
            pltpu.VMEM_SHARED((PLANE + 16,), jnp.float32),
            pltpu.VMEM_SHARED((PLANE + 16,), jnp.float32),
            pltpu.SemaphoreType.DMA,
            pltpu.SemaphoreType.DMA,
            pltpu.SemaphoreType.DMA,
            pltpu.SemaphoreType.DMA,
        ],
    )(_sc_kernel)

    out = kern(x2, img_flat, px, py, ix, iy)
    return out.reshape(C, X_H, X_W)


# speedup vs baseline: 41.4962x; 1.0055x over previous
import functools

import jax
import jax.numpy as jnp
from jax import lax
from jax.experimental import pallas as pl
from jax.experimental.pallas import tpu as pltpu
from jax.experimental.pallas import tpu_sc as plsc

C = 64
L = 100000
X_H = 512
X_W = 512
IMG_H = 128
IMG_W = 2048

N_TILES = 16
P = 6272
LP = N_TILES * P
ROWS = P // 16
PLANE = X_H * X_W
SLICE = PLANE // N_TILES
CH_PER_CORE = C // 2


def _sc_kernel(x_hbm, img_hbm, px_hbm, py_hbm, ix_hbm, iy_hbm, out_hbm,
               ta, tb, pidx_v, didx_v, vals0, vals1, plane0, plane1,
               gsem0, gsem1, psem, osem):
    cid = lax.axis_index("c")
    sid = lax.axis_index("s")

    pltpu.sync_copy(px_hbm.at[sid], ta)
    pltpu.sync_copy(py_hbm.at[sid], tb)

    def body_p(j, carry):
        s = pl.ds(j * 16, 16)
        pidx_v[s] = tb[s] * IMG_W + ta[s]
        return carry

    lax.fori_loop(0, ROWS, body_p, 0)

    pltpu.sync_copy(ix_hbm.at[sid], ta)
    pltpu.sync_copy(iy_hbm.at[sid], tb)

    def body_d(j, carry):
        s = pl.ds(j * 16, 16)
        didx_v[s] = ta[s] * X_W + tb[s]
        return carry

    lax.fori_loop(0, ROWS, body_d, 0)

    def chan(k):
        return cid * CH_PER_CORE + k

    def gather_src(k):
        return img_hbm.at[pl.ds(chan(k) * PLANE, PLANE)].at[pidx_v]

    def plane_slice(buf):
        return buf.at[pl.ds(sid * SLICE, SLICE)]

    def x_slice(k):
        return x_hbm.at[chan(k), pl.ds(sid * SLICE, SLICE)]

    def out_slice(k):
        return out_hbm.at[chan(k), pl.ds(sid * SLICE, SLICE)]

    pltpu.async_copy(gather_src(0), vals0, gsem0)
    pltpu.async_copy(gather_src(1), vals1, gsem1)
    pltpu.sync_copy(x_slice(0), plane_slice(plane0))
    plsc.subcore_barrier()

    def chan_body(k, carry):
        def run(b_static):
            vals = vals0 if b_static == 0 else vals1
            gsem = gsem0 if b_static == 0 else gsem1
            plane = plane0 if b_static == 0 else plane1
            planen = plane1 if b_static == 0 else plane0

            @pl.when(k >= 1)
            def _drain_out():
                pltpu.make_async_copy(plane_slice(planen), out_slice(k - 1),
                                      osem).wait()

            @pl.when(k < CH_PER_CORE - 1)
            def _stage_next():
                pltpu.async_copy(x_slice(k + 1), plane_slice(planen), psem)

            pltpu.make_async_copy(gather_src(k), vals, gsem).wait()
            pltpu.sync_copy(vals, plane.at[didx_v], add=True)

            @pl.when(k < CH_PER_CORE - 2)
            def _prefetch_gather():
                pltpu.async_copy(gather_src(k + 2), vals, gsem)

            @pl.when(k < CH_PER_CORE - 1)
            def _wait_stage():
                pltpu.make_async_copy(x_slice(k + 1), plane_slice(planen),
                                      psem).wait()

            plsc.subcore_barrier()
            pltpu.async_copy(plane_slice(plane), out_slice(k), osem)

        @pl.when(k % 2 == 0)
        def _():
            run(0)

        @pl.when(k % 2 == 1)
        def _():
            run(1)

        return carry

    lax.fori_loop(0, CH_PER_CORE, chan_body, 0)

    b_last = (CH_PER_CORE - 1) % 2
    last_plane = plane1 if b_last else plane0
    pltpu.make_async_copy(plane_slice(last_plane),
                          out_slice(CH_PER_CORE - 1), osem).wait()


def kernel(x, img, index_x, index_y, proj_x, proj_y):
    pad = LP - L
    px = jnp.concatenate([proj_x, jnp.zeros((pad,), jnp.int32)])
    py = jnp.concatenate([proj_y, jnp.zeros((pad,), jnp.int32)])
    ix = jnp.concatenate([index_x.reshape(-1), jnp.full((pad,), X_H, jnp.int32)])
    iy = jnp.concatenate([index_y.reshape(-1), jnp.zeros((pad,), jnp.int32)])
    px = px.reshape(N_TILES, P)
    py = py.reshape(N_TILES, P)
    ix = ix.reshape(N_TILES, P)
    iy = iy.reshape(N_TILES, P)

    x2 = x.reshape(C, PLANE)
    img_flat = img.reshape(-1)

    mesh = plsc.VectorSubcoreMesh(core_axis_name="c", subcore_axis_name="s")
    kern = functools.partial(
        pl.kernel,
        out_type=jax.ShapeDtypeStruct((C, PLANE), jnp.float32),
        mesh=mesh,
        scratch_types=[
            pltpu.VMEM((P,), jnp.int32),
            pltpu.VMEM((P,), jnp.int32),
            pltpu.VMEM((P,), jnp.int32),
            pltpu.VMEM((P,), jnp.int32),
            pltpu.VMEM((P,), jnp.float32),
            pltpu.VMEM((P,), jnp.float32),
            pltpu.VMEM_SHARED((PLANE + 16,), jnp.float32),
            pltpu.VMEM_SHARED((PLANE + 16,), jnp.float32),
            pltpu.SemaphoreType.DMA,
            pltpu.SemaphoreType.DMA,
            pltpu.SemaphoreType.DMA,
            pltpu.SemaphoreType.DMA,
        ],
    )(_sc_kernel)

    out = kern(x2, img_flat, px, py, ix, iy)
    return out.reshape(C, X_H, X_W)


# speedup vs baseline: 54.4721x; 1.3127x over previous
import functools

import jax
import jax.numpy as jnp
from jax import lax
from jax.experimental import pallas as pl
from jax.experimental.pallas import tpu as pltpu
from jax.experimental.pallas import tpu_sc as plsc

C = 64
L = 100000
X_H = 512
X_W = 512
IMG_H = 128
IMG_W = 2048

N_TILES = 16
P = 6272
LP = N_TILES * P
ROWS = P // 16
PLANE = X_H * X_W
R_PER_TILE = X_H // N_TILES
CH_PER_CORE = C // 2


def _sc_kernel(x_hbm, img_hbm, px_hbm, py_hbm, ix_hbm, iy_hbm, out_hbm,
               ta, tb, pidx_v, didx_v, vals0, vals1, plane0, plane1,
               gsem0, gsem1, ssem, psem, osem):
    cid = lax.axis_index("c")
    sid = lax.axis_index("s")

    pltpu.sync_copy(px_hbm.at[sid], ta)
    pltpu.sync_copy(py_hbm.at[sid], tb)

    def body_p(j, carry):
        s = pl.ds(j * 16, 16)
        pidx_v[s] = tb[s] * IMG_W + ta[s]
        return carry

    lax.fori_loop(0, ROWS, body_p, 0)

    pltpu.sync_copy(ix_hbm.at[sid], ta)
    pltpu.sync_copy(iy_hbm.at[sid], tb)

    def body_d(j, carry):
        s = pl.ds(j * 16, 16)
        didx_v[s] = ta[s] * X_W + tb[s]
        return carry

    lax.fori_loop(0, ROWS, body_d, 0)

    row0 = sid * R_PER_TILE

    def chan(k):
        return cid * CH_PER_CORE + k

    def gather_src(k):
        return img_hbm.at[pl.ds(chan(k) * PLANE, PLANE)].at[pidx_v]

    def rows_in(k, plane, sem):
        def body(r, carry):
            pltpu.async_copy(x_hbm.at[chan(k), row0 + r, :],
                             plane.at[pl.ds((row0 + r) * X_W, X_W)], sem)
            return carry
        lax.fori_loop(0, R_PER_TILE, body, 0)

    def rows_in_wait(k, plane, sem):
        def body(r, carry):
            pltpu.make_async_copy(x_hbm.at[chan(k), row0 + r, :],
                                  plane.at[pl.ds((row0 + r) * X_W, X_W)],
                                  sem).wait()
            return carry
        lax.fori_loop(0, R_PER_TILE, body, 0)

    def rows_out(k, plane):
        def body(r, carry):
            pltpu.async_copy(plane.at[pl.ds((row0 + r) * X_W, X_W)],
                             out_hbm.at[chan(k), row0 + r, :], osem)
            return carry
        lax.fori_loop(0, R_PER_TILE, body, 0)

    def rows_out_wait(k, plane):
        def body(r, carry):
            pltpu.make_async_copy(plane.at[pl.ds((row0 + r) * X_W, X_W)],
                                  out_hbm.at[chan(k), row0 + r, :],
                                  osem).wait()
            return carry
        lax.fori_loop(0, R_PER_TILE, body, 0)

    pltpu.async_copy(gather_src(0), vals0, gsem0)
    pltpu.async_copy(gather_src(1), vals1, gsem1)
    rows_in(0, plane0, psem)
    rows_in_wait(0, plane0, psem)
    plsc.subcore_barrier()

    def chan_body(k, carry):
        def run(b_static):
            vals = vals0 if b_static == 0 else vals1
            gsem = gsem0 if b_static == 0 else gsem1
            plane = plane0 if b_static == 0 else plane1
            planen = plane1 if b_static == 0 else plane0

            pltpu.make_async_copy(gather_src(k), vals, gsem).wait()
            pltpu.async_copy(vals, plane.at[didx_v], ssem, add=True)

            @pl.when(k >= 1)
            def _drain_out():
                rows_out_wait(k - 1, planen)

            @pl.when(k < CH_PER_CORE - 1)
            def _stage_next():
                rows_in(k + 1, planen, psem)
                rows_in_wait(k + 1, planen, psem)

            pltpu.make_async_copy(vals, plane.at[didx_v], ssem).wait()

            @pl.when(k < CH_PER_CORE - 2)
            def _prefetch_gather():
                pltpu.async_copy(gather_src(k + 2), vals, gsem)

            plsc.subcore_barrier()
            rows_out(k, plane)

        @pl.when(k % 2 == 0)
        def _():
            run(0)

        @pl.when(k % 2 == 1)
        def _():
            run(1)

        return carry

    lax.fori_loop(0, CH_PER_CORE, chan_body, 0)

    b_last = (CH_PER_CORE - 1) % 2
    last_plane = plane1 if b_last else plane0
    rows_out_wait(CH_PER_CORE - 1, last_plane)


def kernel(x, img, index_x, index_y, proj_x, proj_y):
    pad = LP - L
    px = jnp.concatenate([proj_x, jnp.zeros((pad,), jnp.int32)])
    py = jnp.concatenate([proj_y, jnp.zeros((pad,), jnp.int32)])
    ix = jnp.concatenate([index_x.reshape(-1), jnp.full((pad,), X_H, jnp.int32)])
    iy = jnp.concatenate([index_y.reshape(-1), jnp.zeros((pad,), jnp.int32)])
    px = px.reshape(N_TILES, P)
    py = py.reshape(N_TILES, P)
    ix = ix.reshape(N_TILES, P)
    iy = iy.reshape(N_TILES, P)

    x3 = x.reshape(C, X_H, X_W)
    img_flat = img.reshape(-1)

    mesh = plsc.VectorSubcoreMesh(core_axis_name="c", subcore_axis_name="s")
    kern = functools.partial(
        pl.kernel,
        out_type=jax.ShapeDtypeStruct((C, X_H, X_W), jnp.float32),
        mesh=mesh,
        scratch_types=[
            pltpu.VMEM((P,), jnp.int32),
            pltpu.VMEM((P,), jnp.int32),
            pltpu.VMEM((P,), jnp.int32),
            pltpu.VMEM((P,), jnp.int32),
            pltpu.VMEM((P,), jnp.float32),
            pltpu.VMEM((P,), jnp.float32),
            pltpu.VMEM_SHARED((PLANE + 16,), jnp.float32),
            pltpu.VMEM_SHARED((PLANE + 16,), jnp.float32),
            pltpu.SemaphoreType.DMA,
            pltpu.SemaphoreType.DMA,
            pltpu.SemaphoreType.DMA,
            pltpu.SemaphoreType.DMA,
            pltpu.SemaphoreType.DMA,
        ],
    )(_sc_kernel)

    return kern(x3, img_flat, px, py, ix, iy)
